# baseline (speedup 1.0000x reference)
import jax
import jax.numpy as jnp
from jax import lax
from jax.experimental import pallas as pl
from jax.experimental.pallas import tpu as pltpu, tpu_sc as plsc

B = 16384
D = 16
NV = 1000000
NC, NS = 2, 16
NW = NC * NS
BPW = B // NW
ICH = 128
NJ = BPW // ICH
NCH = BPW // 16


def _sc_body(uids, iids, wot, wh, wit, out, ids_u, ids_v, idx_u, idx_v,
             ut, vt, whv, outv, sem_u, sem_v):
    wid = lax.axis_index("s") * NC + lax.axis_index("c")
    base = wid * BPW

    pltpu.sync_copy(uids.at[pl.ds(base, BPW)], ids_u)
    pltpu.sync_copy(iids.at[pl.ds(base, BPW)], ids_v)
    pltpu.sync_copy(wh, whv)

    for d in range(D):
        doff = jnp.full((16,), d * NV, jnp.int32)
        for j in range(NJ):
            for t in range(ICH // 16):
                iu = ids_u[pl.ds(j * ICH + t * 16, 16)]
                iv = ids_v[pl.ds(j * ICH + t * 16, 16)]
                idx_u[d, j, pl.ds(t * 16, 16)] = iu + doff
                idx_v[d, j, pl.ds(t * 16, 16)] = iv + doff

    copies = []
    for d in range(D):
        for j in range(NJ):
            copies.append(pltpu.async_copy(
                wot.at[idx_u.at[d, j]], ut.at[d, pl.ds(j * ICH, ICH)], sem_u))
            copies.append(pltpu.async_copy(
                wit.at[idx_v.at[d, j]], vt.at[d, pl.ds(j * ICH, ICH)], sem_v))
    for c in copies:
        c.wait()

    wh_rows = [whv[pl.ds(d * 16, 16)] for d in range(D)]

    def chunk(c, carry):
        s = c * 16
        vcols = [vt[e, pl.ds(s, 16)] for e in range(D)]
        acc = jnp.zeros((16,), jnp.float32)
        for d in range(D):
            t = jnp.zeros((16,), jnp.float32)
            for e in range(D):
                t = t + wh_rows[d][e] * vcols[e]
            acc = acc + ut[d, pl.ds(s, 16)] * t
        outv[pl.ds(s, 16)] = acc
        return carry

    lax.fori_loop(0, NCH, chunk, 0)
    pltpu.sync_copy(outv, out.at[pl.ds(base, BPW)])


def kernel(user_ids, item_ids, W_o, W_h, W_i):
    wot = W_o.T.reshape(D * NV)
    wit = W_i.T.reshape(D * NV)
    wh1 = W_h.reshape(D * D)
    mesh = plsc.VectorSubcoreMesh(core_axis_name="c", subcore_axis_name="s")
    f = pl.kernel(
        _sc_body,
        out_type=jax.ShapeDtypeStruct((B,), jnp.float32),
        mesh=mesh,
        compiler_params=pltpu.CompilerParams(needs_layout_passes=False),
        scratch_types=[
            pltpu.VMEM((BPW,), jnp.int32),
            pltpu.VMEM((BPW,), jnp.int32),
            pltpu.VMEM((D, NJ, ICH), jnp.int32),
            pltpu.VMEM((D, NJ, ICH), jnp.int32),
            pltpu.VMEM((D, BPW), jnp.float32),
            pltpu.VMEM((D, BPW), jnp.float32),
            pltpu.VMEM((D * D,), jnp.float32),
            pltpu.VMEM((BPW,), jnp.float32),
            pltpu.SemaphoreType.DMA,
            pltpu.SemaphoreType.DMA,
        ],
    )
    return f(user_ids, item_ids, wot, wh1, wit)


# speedup vs baseline: 9.7845x; 9.7845x over previous
import jax
import jax.numpy as jnp
from jax import lax
from jax.experimental import pallas as pl
from jax.experimental.pallas import tpu as pltpu, tpu_sc as plsc

B = 16384
D = 16
NV = 1000000
P = 1000064
NC, NS = 2, 16
NW = NC * NS

PW = 4608
NPF = NV // PW
TAIL = NV - NPF * PW
NPIECE = NPF + 1
NTASK = 2 * NPIECE
TPW = (NTASK + NW - 1) // NW

BPW = B // NW
ICH = 128
NJ = BPW // ICH
NCH = BPW // 16


def _detile_body(wot, wit, tails, fo, fi, stg, outb, tstg, sem_in, sem_out):
    wid = lax.axis_index("s") * NC + lax.axis_index("c")

    def do_piece(src, dst, piece):
        c0 = piece * PW
        for g in range(2):
            pltpu.async_copy(
                src.at[pl.ds(g * 8, 8), pl.ds(c0, PW)], stg, sem_in).wait()

            def row16(c16, carry):
                sl = pl.ds(c16 * 16, 16)
                for s in range(8):
                    outb[s, sl] = stg[s, sl]
                return carry

            lax.fori_loop(0, PW // 16, row16, 0)
            cps = []
            for s in range(8):
                d = g * 8 + s
                cps.append(pltpu.async_copy(
                    outb.at[s, pl.ds(0, PW)],
                    dst.at[pl.ds(d * P + c0, PW)], sem_out))
            for c in cps:
                c.wait()

    def do_tail(tbl_idx, dst):
        pltpu.async_copy(tails.at[tbl_idx], tstg, sem_in).wait()

        def row16(c16, carry):
            sl = pl.ds(c16 * 16, 16)
            for s in range(16):
                outb[s % 8, pl.ds((s // 8) * 128 + c16 * 16, 16)] = tstg[s, sl]
            return carry

        lax.fori_loop(0, TAIL // 16, row16, 0)
        cps = []
        for d in range(D):
            cps.append(pltpu.async_copy(
                outb.at[d % 8, pl.ds((d // 8) * 128, TAIL)],
                dst.at[pl.ds(d * P + NPF * PW, TAIL)], sem_out))
        for c in cps:
            c.wait()

    def task_body(k, carry):
        task = wid + NW * k

        @pl.when(task < NTASK)
        def _():
            tbl = task % 2
            piece = task // 2

            @pl.when(piece < NPF)
            def _():
                @pl.when(tbl == 0)
                def _():
                    do_piece(wot, fo, piece)

                @pl.when(tbl == 1)
                def _():
                    do_piece(wit, fi, piece)

            @pl.when(piece == NPF)
            def _():
                @pl.when(tbl == 0)
                def _():
                    do_tail(0, fo)

                @pl.when(tbl == 1)
                def _():
                    do_tail(1, fi)

        return carry

    lax.fori_loop(0, TPW, task_body, 0)


def _gather_body(uids, iids, fo, wh, fi, out, ids_u, ids_v, idx_u, idx_v,
                 ut, vt, whv, outv, sem_u, sem_v):
    wid = lax.axis_index("s") * NC + lax.axis_index("c")
    base = wid * BPW

    pltpu.sync_copy(uids.at[pl.ds(base, BPW)], ids_u)
    pltpu.sync_copy(iids.at[pl.ds(base, BPW)], ids_v)
    pltpu.sync_copy(wh, whv)

    for d in range(D):
        doff = jnp.full((16,), d * P, jnp.int32)
        for j in range(NJ):
            for t in range(ICH // 16):
                iu = ids_u[pl.ds(j * ICH + t * 16, 16)]
                iv = ids_v[pl.ds(j * ICH + t * 16, 16)]
                idx_u[d, j, pl.ds(t * 16, 16)] = iu + doff
                idx_v[d, j, pl.ds(t * 16, 16)] = iv + doff

    copies = []
    for d in range(D):
        for j in range(NJ):
            copies.append(pltpu.async_copy(
                fo.at[idx_u.at[d, j]], ut.at[d, pl.ds(j * ICH, ICH)], sem_u))
            copies.append(pltpu.async_copy(
                fi.at[idx_v.at[d, j]], vt.at[d, pl.ds(j * ICH, ICH)], sem_v))
    for c in copies:
        c.wait()

    wh_rows = [whv[pl.ds(d * 16, 16)] for d in range(D)]

    def chunk(c, carry):
        s = c * 16
        vcols = [vt[e, pl.ds(s, 16)] for e in range(D)]
        acc = jnp.zeros((16,), jnp.float32)
        for d in range(D):
            t = jnp.zeros((16,), jnp.float32)
            for e in range(D):
                t = t + wh_rows[d][e] * vcols[e]
            acc = acc + ut[d, pl.ds(s, 16)] * t
        outv[pl.ds(s, 16)] = acc
        return carry

    lax.fori_loop(0, NCH, chunk, 0)
    pltpu.sync_copy(outv, out.at[pl.ds(base, BPW)])


def kernel(user_ids, item_ids, W_o, W_h, W_i):
    mesh = plsc.VectorSubcoreMesh(core_axis_name="c", subcore_axis_name="s")

    detile = pl.kernel(
        _detile_body,
        out_type=(jax.ShapeDtypeStruct((D * P,), jnp.float32),
                  jax.ShapeDtypeStruct((D * P,), jnp.float32)),
        mesh=mesh,
        scratch_types=[
            pltpu.VMEM((8, PW), jnp.float32),
            pltpu.VMEM((8, PW), jnp.float32),
            pltpu.VMEM((D, 128), jnp.float32),
            pltpu.SemaphoreType.DMA,
            pltpu.SemaphoreType.DMA,
        ],
    )
    tails = jnp.stack([
        jnp.pad(W_o[NPF * PW:].T, ((0, 0), (0, 128 - TAIL))),
        jnp.pad(W_i[NPF * PW:].T, ((0, 0), (0, 128 - TAIL))),
    ])
    fo, fi = detile(W_o.T, W_i.T, tails)

    gather = pl.kernel(
        _gather_body,
        out_type=jax.ShapeDtypeStruct((B,), jnp.float32),
        mesh=mesh,
        compiler_params=pltpu.CompilerParams(
            needs_layout_passes=False, use_tc_tiling_on_sc=False),
        scratch_types=[
            pltpu.VMEM((BPW,), jnp.int32),
            pltpu.VMEM((BPW,), jnp.int32),
            pltpu.VMEM((D, NJ, ICH), jnp.int32),
            pltpu.VMEM((D, NJ, ICH), jnp.int32),
            pltpu.VMEM((D, BPW), jnp.float32),
            pltpu.VMEM((D, BPW), jnp.float32),
            pltpu.VMEM((D * D,), jnp.float32),
            pltpu.VMEM((BPW,), jnp.float32),
            pltpu.SemaphoreType.DMA,
            pltpu.SemaphoreType.DMA,
        ],
    )
    return gather(user_ids, item_ids, fo, W_h.reshape(D * D), fi)
